# Initial kernel scaffold; baseline (speedup 1.0000x reference)
import jax
import jax.numpy as jnp
from jax.experimental import pallas as pl


def kernel(x, loc, edge_index, inter_ids, W_fc, W_G, embed_table, boundaries):
    raise NotImplementedError("write your pallas kernel here")


# speedup vs baseline: 8.2782x; 8.2782x over previous
import functools

import jax
import jax.numpy as jnp
from jax import lax
from jax.experimental import pallas as pl
from jax.experimental.pallas import tpu as pltpu
from jax.experimental.pallas import tpu_sc as plsc

N = 10000
E = 160000
D_IN = 256
D_OUT = 256
H = 128
NT = 16
EPT = E // NT
CH = 80
NCHUNK = EPT // CH
RPT = 632
RPT_LAST = N - (NT - 1) * RPT

_mesh = plsc.VectorSubcoreMesh(core_axis_name="c", subcore_axis_name="s")


@functools.partial(
    pl.kernel,
    out_type=jax.ShapeDtypeStruct((2, N), jnp.float32),
    mesh=_mesh,
    scratch_types=[
        pltpu.VMEM((CH,), jnp.int32),
        pltpu.VMEM((CH,), jnp.float32),
        pltpu.VMEM_SHARED((N,), jnp.float32),
    ],
)
def _degree_kernel(src_hbm, dst_hbm, zeros_hbm, deg_hbm, idx_v, ones_v, acc_sh):
    c = lax.axis_index("c")
    s = lax.axis_index("s")

    @pl.when(s == 0)
    def _():
        pltpu.sync_copy(zeros_hbm, acc_sh)

    for j in range(CH // 16):
        ones_v[pl.ds(j * 16, 16)] = jnp.full((16,), 1.0, jnp.float32)
    plsc.subcore_barrier()

    base = s * EPT

    def run(edge_hbm):
        def body(i, carry):
            pltpu.sync_copy(edge_hbm.at[pl.ds(base + i * CH, CH)], idx_v)
            pltpu.sync_copy(ones_v, acc_sh.at[idx_v], add=True)
            return carry

        lax.fori_loop(0, NCHUNK, body, 0)

    @pl.when(c == 0)
    def _():
        run(src_hbm)

    @pl.when(c == 1)
    def _():
        run(dst_hbm)

    plsc.subcore_barrier()

    @pl.when(s == 0)
    def _():
        pltpu.sync_copy(acc_sh, deg_hbm.at[c])


def _scale_body(x_ref, degt_ref, gl_ref, gr_ref, d0_ref):
    dt = degt_ref[...]
    d2 = lax.rsqrt(jnp.maximum(dt[:, 0:1], 1.0))
    d0 = lax.rsqrt(jnp.maximum(dt[:, 1:2], 1.0))
    g = x_ref[...] * d2
    gl_ref[...] = g[:, :H]
    gr_ref[...] = g[:, H:]
    d0_ref[...] = d0


def _scale_call(x, degT):
    R = 2000
    grid = N // R
    return pl.pallas_call(
        _scale_body,
        grid=(grid,),
        in_specs=[
            pl.BlockSpec((R, D_IN), lambda i: (i, 0)),
            pl.BlockSpec((R, 2), lambda i: (i, 0)),
        ],
        out_specs=[
            pl.BlockSpec((R, H), lambda i: (i, 0)),
            pl.BlockSpec((R, H), lambda i: (i, 0)),
            pl.BlockSpec((R, 1), lambda i: (i, 0)),
        ],
        out_shape=[
            jax.ShapeDtypeStruct((N, H), jnp.float32),
            jax.ShapeDtypeStruct((N, H), jnp.float32),
            jax.ShapeDtypeStruct((N, 1), jnp.float32),
        ],
    )(x, degT)


@functools.partial(
    pl.kernel,
    out_type=[
        jax.ShapeDtypeStruct((N, H), jnp.float32),
        jax.ShapeDtypeStruct((N, H), jnp.float32),
    ],
    mesh=_mesh,
    scratch_types=[
        pltpu.VMEM((CH,), jnp.int32),
        pltpu.VMEM((CH,), jnp.int32),
        pltpu.VMEM((CH, H), jnp.float32),
        pltpu.VMEM_SHARED((N, H), jnp.float32),
        pltpu.SemaphoreType.DMA,
    ],
)
def _gather_scatter_kernel(gl_hbm, gr_hbm, src_hbm, dst_hbm, zeros_hbm,
                           aggl_hbm, aggr_hbm,
                           src_v, dst_v, rows_v, acc_sh, sem):
    c = lax.axis_index("c")
    s = lax.axis_index("s")

    def _rowslice(ref):
        start = pl.multiple_of(s * RPT, 8)
        return ref.at[pl.ds(start, RPT)]

    def _rowslice_last(ref):
        return ref.at[pl.ds((NT - 1) * RPT, RPT_LAST)]

    @pl.when(s < NT - 1)
    def _():
        pltpu.sync_copy(_rowslice(zeros_hbm), _rowslice(acc_sh))

    @pl.when(s == NT - 1)
    def _():
        pltpu.sync_copy(_rowslice_last(zeros_hbm), _rowslice_last(acc_sh))

    plsc.subcore_barrier()

    base = s * EPT

    def run(g_hbm):
        def body(i, carry):
            off = base + i * CH
            pltpu.sync_copy(src_hbm.at[pl.ds(off, CH)], src_v)
            pltpu.sync_copy(dst_hbm.at[pl.ds(off, CH)], dst_v)
            pltpu.async_copy(g_hbm.at[src_v], rows_v, sem).wait()
            pltpu.sync_copy(rows_v, acc_sh.at[dst_v], add=True)
            return carry

        lax.fori_loop(0, NCHUNK, body, 0)

    @pl.when(c == 0)
    def _():
        run(gl_hbm)

    @pl.when(c == 1)
    def _():
        run(gr_hbm)

    plsc.subcore_barrier()

    def writeout(agg_hbm):
        @pl.when(s < NT - 1)
        def _():
            pltpu.sync_copy(_rowslice(acc_sh), _rowslice(agg_hbm))

        @pl.when(s == NT - 1)
        def _():
            pltpu.sync_copy(_rowslice_last(acc_sh), _rowslice_last(agg_hbm))

    @pl.when(c == 0)
    def _():
        writeout(aggl_hbm)

    @pl.when(c == 1)
    def _():
        writeout(aggr_hbm)


def _matmul_body(al_ref, ar_ref, w_ref, d0_ref, ft_ref):
    w = w_ref[...]
    acc = lax.dot_general(al_ref[...], w[:, :H], (((1,), (1,)), ((), ())),
                          preferred_element_type=jnp.float32)
    acc = acc + lax.dot_general(ar_ref[...], w[:, H:], (((1,), (1,)), ((), ())),
                                preferred_element_type=jnp.float32)
    ft_ref[...] = acc * d0_ref[...]


def _matmul_call(aggl, aggr, W_fc, d0):
    R = 2000
    grid = N // R
    return pl.pallas_call(
        _matmul_body,
        grid=(grid,),
        in_specs=[
            pl.BlockSpec((R, H), lambda i: (i, 0)),
            pl.BlockSpec((R, H), lambda i: (i, 0)),
            pl.BlockSpec((D_OUT, D_IN), lambda i: (0, 0)),
            pl.BlockSpec((R, 1), lambda i: (i, 0)),
        ],
        out_specs=pl.BlockSpec((R, D_OUT), lambda i: (i, 0)),
        out_shape=jax.ShapeDtypeStruct((N, D_OUT), jnp.float32),
    )(aggl, aggr, W_fc, d0)


def kernel(x, loc, edge_index, inter_ids, W_fc, W_G, embed_table, boundaries):
    src = edge_index[0]
    dst = edge_index[1]
    zeros_n = jnp.zeros((N,), jnp.float32)
    zeros_nd = jnp.zeros((N, H), jnp.float32)

    deg = _degree_kernel(src, dst, zeros_n)
    gl, gr, d0 = _scale_call(x, deg.T)
    aggl, aggr = _gather_scatter_kernel(gl, gr, src, dst, zeros_nd)
    return _matmul_call(aggl, aggr, W_fc, d0)


# speedup vs baseline: 14.8834x; 1.7979x over previous
import functools

import jax
import jax.numpy as jnp
from jax import lax
from jax.experimental import pallas as pl
from jax.experimental.pallas import tpu as pltpu
from jax.experimental.pallas import tpu_sc as plsc

N = 10000
E = 160000
D_IN = 256
D_OUT = 256
H = 128
NT = 16
EPT = E // NT
CH = 80
NCHUNK = EPT // CH
CHC = 40
NCHUNKC = EPT // CHC
RPT = 632
RPT_LAST = N - (NT - 1) * RPT

_mesh = plsc.VectorSubcoreMesh(core_axis_name="c", subcore_axis_name="s")


@functools.partial(
    pl.kernel,
    out_type=jax.ShapeDtypeStruct((2, N), jnp.float32),
    mesh=_mesh,
    scratch_types=[
        pltpu.VMEM((CH,), jnp.int32),
        pltpu.VMEM((CH,), jnp.float32),
        pltpu.VMEM_SHARED((N,), jnp.float32),
    ],
)
def _degree_kernel(src_hbm, dst_hbm, zeros_hbm, deg_hbm, idx_v, ones_v, acc_sh):
    c = lax.axis_index("c")
    s = lax.axis_index("s")

    @pl.when(s == 0)
    def _():
        pltpu.sync_copy(zeros_hbm, acc_sh)

    for j in range(CH // 16):
        ones_v[pl.ds(j * 16, 16)] = jnp.full((16,), 1.0, jnp.float32)
    plsc.subcore_barrier()

    base = s * EPT

    def run(edge_hbm):
        def body(i, carry):
            pltpu.sync_copy(edge_hbm.at[pl.ds(base + i * CH, CH)], idx_v)
            pltpu.sync_copy(ones_v, acc_sh.at[idx_v], add=True)
            return carry

        lax.fori_loop(0, NCHUNK, body, 0)

    @pl.when(c == 0)
    def _():
        run(src_hbm)

    @pl.when(c == 1)
    def _():
        run(dst_hbm)

    plsc.subcore_barrier()

    @pl.when(s == 0)
    def _():
        pltpu.sync_copy(acc_sh, deg_hbm.at[c])


def _scale_body(x_ref, degt_ref, gl_ref, gr_ref, d0_ref):
    dt = degt_ref[...]
    d2 = lax.rsqrt(jnp.maximum(dt[:, 0:1], 1.0))
    d0 = lax.rsqrt(jnp.maximum(dt[:, 1:2], 1.0))
    g = x_ref[...] * d2
    gl_ref[...] = g[:, :H]
    gr_ref[...] = g[:, H:]
    d0_ref[...] = d0


def _scale_call(x, degT):
    R = 2000
    grid = N // R
    return pl.pallas_call(
        _scale_body,
        grid=(grid,),
        in_specs=[
            pl.BlockSpec((R, D_IN), lambda i: (i, 0)),
            pl.BlockSpec((R, 2), lambda i: (i, 0)),
        ],
        out_specs=[
            pl.BlockSpec((R, H), lambda i: (i, 0)),
            pl.BlockSpec((R, H), lambda i: (i, 0)),
            pl.BlockSpec((R, 1), lambda i: (i, 0)),
        ],
        out_shape=[
            jax.ShapeDtypeStruct((N, H), jnp.float32),
            jax.ShapeDtypeStruct((N, H), jnp.float32),
            jax.ShapeDtypeStruct((N, 1), jnp.float32),
        ],
    )(x, degT)


K = 2
NG = NCHUNKC // K


@functools.partial(
    pl.kernel,
    out_type=[
        jax.ShapeDtypeStruct((N, H), jnp.float32),
        jax.ShapeDtypeStruct((N, H), jnp.float32),
    ],
    mesh=_mesh,
    scratch_types=[
        pltpu.VMEM((EPT,), jnp.int32),
        pltpu.VMEM((2, K, CHC), jnp.int32),
        pltpu.VMEM((2, K, CHC, H), jnp.float32),
        pltpu.VMEM_SHARED((N, H), jnp.float32),
        pltpu.SemaphoreType.DMA((2,)),
        pltpu.SemaphoreType.DMA((2,)),
        pltpu.SemaphoreType.DMA((2,)),
    ],
)
def _gather_scatter_kernel(gl_hbm, gr_hbm, src_hbm, dst_hbm, zeros_hbm,
                           aggl_hbm, aggr_hbm,
                           src_all, dst_v, rows_v, acc_sh, sem_i, sem_g, sem_s):
    c = lax.axis_index("c")
    s = lax.axis_index("s")

    def _rowslice(ref):
        start = pl.multiple_of(s * RPT, 8)
        return ref.at[pl.ds(start, RPT)]

    def _rowslice_last(ref):
        return ref.at[pl.ds((NT - 1) * RPT, RPT_LAST)]

    @pl.when(s < NT - 1)
    def _():
        pltpu.sync_copy(_rowslice(zeros_hbm), _rowslice(acc_sh))

    @pl.when(s == NT - 1)
    def _():
        pltpu.sync_copy(_rowslice_last(zeros_hbm), _rowslice_last(acc_sh))

    plsc.subcore_barrier()

    base = s * EPT

    def run(g_hbm):
        pltpu.sync_copy(src_hbm.at[pl.ds(base, EPT)], src_all)

        def issue_group(g, p):
            for b in range(K):
                off = g * (K * CHC) + b * CHC
                pltpu.async_copy(dst_hbm.at[pl.ds(base + off, CHC)],
                                 dst_v.at[p, b], sem_i.at[p])
                pltpu.async_copy(g_hbm.at[src_all.at[pl.ds(off, CHC)]],
                                 rows_v.at[p, b], sem_g.at[p])

        def drain_scatters(p):
            for b in range(K):
                pltpu.make_async_copy(rows_v.at[p, b],
                                      acc_sh.at[dst_v.at[p, b]],
                                      sem_s.at[p]).wait()

        issue_group(0, 0)

        def body(g, carry):
            p = lax.rem(g, 2)
            q = 1 - p

            @pl.when(g >= 1)
            def _():
                drain_scatters(q)

            @pl.when(g + 1 < NG)
            def _():
                issue_group(g + 1, q)

            for b in range(K):
                pltpu.make_async_copy(g_hbm.at[src_all.at[pl.ds(0, CHC)]],
                                      rows_v.at[p, b], sem_g.at[p]).wait()
            for b in range(K):
                pltpu.make_async_copy(dst_hbm.at[pl.ds(base, CHC)],
                                      dst_v.at[p, b], sem_i.at[p]).wait()
            for b in range(K):
                pltpu.async_copy(rows_v.at[p, b], acc_sh.at[dst_v.at[p, b]],
                                 sem_s.at[p], add=True)
            return carry

        lax.fori_loop(0, NG, body, 0)
        drain_scatters((NG - 1) % 2)

    @pl.when(c == 0)
    def _():
        run(gl_hbm)

    @pl.when(c == 1)
    def _():
        run(gr_hbm)

    plsc.subcore_barrier()

    def writeout(agg_hbm):
        @pl.when(s < NT - 1)
        def _():
            pltpu.sync_copy(_rowslice(acc_sh), _rowslice(agg_hbm))

        @pl.when(s == NT - 1)
        def _():
            pltpu.sync_copy(_rowslice_last(acc_sh), _rowslice_last(agg_hbm))

    @pl.when(c == 0)
    def _():
        writeout(aggl_hbm)

    @pl.when(c == 1)
    def _():
        writeout(aggr_hbm)


def _matmul_body(al_ref, ar_ref, w_ref, d0_ref, ft_ref):
    w = w_ref[...]
    acc = lax.dot_general(al_ref[...], w[:, :H], (((1,), (1,)), ((), ())),
                          preferred_element_type=jnp.float32)
    acc = acc + lax.dot_general(ar_ref[...], w[:, H:], (((1,), (1,)), ((), ())),
                                preferred_element_type=jnp.float32)
    ft_ref[...] = acc * d0_ref[...]


def _matmul_call(aggl, aggr, W_fc, d0):
    R = 2000
    grid = N // R
    return pl.pallas_call(
        _matmul_body,
        grid=(grid,),
        in_specs=[
            pl.BlockSpec((R, H), lambda i: (i, 0)),
            pl.BlockSpec((R, H), lambda i: (i, 0)),
            pl.BlockSpec((D_OUT, D_IN), lambda i: (0, 0)),
            pl.BlockSpec((R, 1), lambda i: (i, 0)),
        ],
        out_specs=pl.BlockSpec((R, D_OUT), lambda i: (i, 0)),
        out_shape=jax.ShapeDtypeStruct((N, D_OUT), jnp.float32),
    )(aggl, aggr, W_fc, d0)


def kernel(x, loc, edge_index, inter_ids, W_fc, W_G, embed_table, boundaries):
    src = edge_index[0]
    dst = edge_index[1]
    zeros_n = jnp.zeros((N,), jnp.float32)
    zeros_nd = jnp.zeros((N, H), jnp.float32)

    deg = _degree_kernel(src, dst, zeros_n)
    gl, gr, d0 = _scale_call(x, deg.T)
    aggl, aggr = _gather_scatter_kernel(gl, gr, src, dst, zeros_nd)
    return _matmul_call(aggl, aggr, W_fc, d0)


# speedup vs baseline: 19.2896x; 1.2960x over previous
import functools

import jax
import jax.numpy as jnp
from jax import lax
from jax.experimental import pallas as pl
from jax.experimental.pallas import tpu as pltpu
from jax.experimental.pallas import tpu_sc as plsc

N = 10000
E = 160000
D_IN = 256
D_OUT = 256
H = 128
NT = 16
EPT = E // NT
CH = 80
NCHUNK = EPT // CH
CHC = 40
NCHUNKC = EPT // CHC
RPT = 632
RPT_LAST = N - (NT - 1) * RPT

_mesh = plsc.VectorSubcoreMesh(core_axis_name="c", subcore_axis_name="s")


KA = 5
NGA = NCHUNK // KA


@functools.partial(
    pl.kernel,
    out_type=jax.ShapeDtypeStruct((2, N), jnp.float32),
    mesh=_mesh,
    scratch_types=[
        pltpu.VMEM((2, KA, CH), jnp.int32),
        pltpu.VMEM((CH,), jnp.float32),
        pltpu.VMEM_SHARED((N,), jnp.float32),
        pltpu.SemaphoreType.DMA((2,)),
        pltpu.SemaphoreType.DMA((2,)),
    ],
)
def _degree_kernel(src_hbm, dst_hbm, zeros_hbm, deg_hbm, idx_v, ones_v, acc_sh,
                   sem_i, sem_s):
    c = lax.axis_index("c")
    s = lax.axis_index("s")

    @pl.when(s == 0)
    def _():
        pltpu.sync_copy(zeros_hbm, acc_sh)

    for j in range(CH // 16):
        ones_v[pl.ds(j * 16, 16)] = jnp.full((16,), 1.0, jnp.float32)
    plsc.subcore_barrier()

    base = s * EPT

    def run(edge_hbm):
        def issue_group(g, p):
            for b in range(KA):
                off = base + g * (KA * CH) + b * CH
                pltpu.async_copy(edge_hbm.at[pl.ds(off, CH)],
                                 idx_v.at[p, b], sem_i.at[p])

        def drain_scatters(p):
            for b in range(KA):
                pltpu.make_async_copy(ones_v, acc_sh.at[idx_v.at[p, b]],
                                      sem_s.at[p]).wait()

        issue_group(0, 0)

        def body(g, carry):
            p = lax.rem(g, 2)
            q = 1 - p

            @pl.when(g >= 1)
            def _():
                drain_scatters(q)

            @pl.when(g + 1 < NGA)
            def _():
                issue_group(g + 1, q)

            for b in range(KA):
                pltpu.make_async_copy(edge_hbm.at[pl.ds(base, CH)],
                                      idx_v.at[p, b], sem_i.at[p]).wait()
            for b in range(KA):
                pltpu.async_copy(ones_v, acc_sh.at[idx_v.at[p, b]],
                                 sem_s.at[p], add=True)
            return carry

        lax.fori_loop(0, NGA, body, 0)
        drain_scatters((NGA - 1) % 2)

    @pl.when(c == 0)
    def _():
        run(src_hbm)

    @pl.when(c == 1)
    def _():
        run(dst_hbm)

    plsc.subcore_barrier()

    @pl.when(s == 0)
    def _():
        pltpu.sync_copy(acc_sh, deg_hbm.at[c])


def _scale_body(x_ref, degt_ref, gl_ref, gr_ref, d0_ref):
    dt = degt_ref[...]
    d2 = lax.rsqrt(jnp.maximum(dt[:, 0:1], 1.0))
    d0 = lax.rsqrt(jnp.maximum(dt[:, 1:2], 1.0))
    g = x_ref[...] * d2
    gl_ref[...] = g[:, :H]
    gr_ref[...] = g[:, H:]
    d0_ref[...] = d0


def _scale_call(x, degT):
    R = 2000
    grid = N // R
    return pl.pallas_call(
        _scale_body,
        grid=(grid,),
        in_specs=[
            pl.BlockSpec((R, D_IN), lambda i: (i, 0)),
            pl.BlockSpec((R, 2), lambda i: (i, 0)),
        ],
        out_specs=[
            pl.BlockSpec((R, H), lambda i: (i, 0)),
            pl.BlockSpec((R, H), lambda i: (i, 0)),
            pl.BlockSpec((R, 1), lambda i: (i, 0)),
        ],
        out_shape=[
            jax.ShapeDtypeStruct((N, H), jnp.float32),
            jax.ShapeDtypeStruct((N, H), jnp.float32),
            jax.ShapeDtypeStruct((N, 1), jnp.float32),
        ],
    )(x, degT)


K = 2
NG = NCHUNKC // K


@functools.partial(
    pl.kernel,
    out_type=[
        jax.ShapeDtypeStruct((N, H), jnp.float32),
        jax.ShapeDtypeStruct((N, H), jnp.float32),
    ],
    mesh=_mesh,
    scratch_types=[
        pltpu.VMEM((EPT,), jnp.int32),
        pltpu.VMEM((2, K, CHC), jnp.int32),
        pltpu.VMEM((2, K, CHC, H), jnp.float32),
        pltpu.VMEM_SHARED((N, H), jnp.float32),
        pltpu.SemaphoreType.DMA((2,)),
        pltpu.SemaphoreType.DMA((2,)),
        pltpu.SemaphoreType.DMA((2,)),
    ],
)
def _gather_scatter_kernel(gl_hbm, gr_hbm, src_hbm, dst_hbm, zeros_hbm,
                           aggl_hbm, aggr_hbm,
                           src_all, dst_v, rows_v, acc_sh, sem_i, sem_g, sem_s):
    c = lax.axis_index("c")
    s = lax.axis_index("s")

    def _rowslice(ref):
        start = pl.multiple_of(s * RPT, 8)
        return ref.at[pl.ds(start, RPT)]

    def _rowslice_last(ref):
        return ref.at[pl.ds((NT - 1) * RPT, RPT_LAST)]

    @pl.when(s < NT - 1)
    def _():
        pltpu.sync_copy(_rowslice(zeros_hbm), _rowslice(acc_sh))

    @pl.when(s == NT - 1)
    def _():
        pltpu.sync_copy(_rowslice_last(zeros_hbm), _rowslice_last(acc_sh))

    plsc.subcore_barrier()

    base = s * EPT

    def run(g_hbm):
        pltpu.sync_copy(src_hbm.at[pl.ds(base, EPT)], src_all)

        def issue_group(g, p):
            for b in range(K):
                off = g * (K * CHC) + b * CHC
                pltpu.async_copy(dst_hbm.at[pl.ds(base + off, CHC)],
                                 dst_v.at[p, b], sem_i.at[p])
                pltpu.async_copy(g_hbm.at[src_all.at[pl.ds(off, CHC)]],
                                 rows_v.at[p, b], sem_g.at[p])

        def drain_scatters(p):
            for b in range(K):
                pltpu.make_async_copy(rows_v.at[p, b],
                                      acc_sh.at[dst_v.at[p, b]],
                                      sem_s.at[p]).wait()

        issue_group(0, 0)

        def body(g, carry):
            p = lax.rem(g, 2)
            q = 1 - p

            @pl.when(g >= 1)
            def _():
                drain_scatters(q)

            @pl.when(g + 1 < NG)
            def _():
                issue_group(g + 1, q)

            for b in range(K):
                pltpu.make_async_copy(g_hbm.at[src_all.at[pl.ds(0, CHC)]],
                                      rows_v.at[p, b], sem_g.at[p]).wait()
            for b in range(K):
                pltpu.make_async_copy(dst_hbm.at[pl.ds(base, CHC)],
                                      dst_v.at[p, b], sem_i.at[p]).wait()
            for b in range(K):
                pltpu.async_copy(rows_v.at[p, b], acc_sh.at[dst_v.at[p, b]],
                                 sem_s.at[p], add=True)
            return carry

        lax.fori_loop(0, NG, body, 0)
        drain_scatters((NG - 1) % 2)

    @pl.when(c == 0)
    def _():
        run(gl_hbm)

    @pl.when(c == 1)
    def _():
        run(gr_hbm)

    plsc.subcore_barrier()

    def writeout(agg_hbm):
        @pl.when(s < NT - 1)
        def _():
            pltpu.sync_copy(_rowslice(acc_sh), _rowslice(agg_hbm))

        @pl.when(s == NT - 1)
        def _():
            pltpu.sync_copy(_rowslice_last(acc_sh), _rowslice_last(agg_hbm))

    @pl.when(c == 0)
    def _():
        writeout(aggl_hbm)

    @pl.when(c == 1)
    def _():
        writeout(aggr_hbm)


def _matmul_body(al_ref, ar_ref, w_ref, d0_ref, ft_ref):
    w = w_ref[...]
    acc = lax.dot_general(al_ref[...], w[:, :H], (((1,), (1,)), ((), ())),
                          preferred_element_type=jnp.float32)
    acc = acc + lax.dot_general(ar_ref[...], w[:, H:], (((1,), (1,)), ((), ())),
                                preferred_element_type=jnp.float32)
    ft_ref[...] = acc * d0_ref[...]


def _matmul_call(aggl, aggr, W_fc, d0):
    R = 2000
    grid = N // R
    return pl.pallas_call(
        _matmul_body,
        grid=(grid,),
        in_specs=[
            pl.BlockSpec((R, H), lambda i: (i, 0)),
            pl.BlockSpec((R, H), lambda i: (i, 0)),
            pl.BlockSpec((D_OUT, D_IN), lambda i: (0, 0)),
            pl.BlockSpec((R, 1), lambda i: (i, 0)),
        ],
        out_specs=pl.BlockSpec((R, D_OUT), lambda i: (i, 0)),
        out_shape=jax.ShapeDtypeStruct((N, D_OUT), jnp.float32),
    )(aggl, aggr, W_fc, d0)


def kernel(x, loc, edge_index, inter_ids, W_fc, W_G, embed_table, boundaries):
    src = edge_index[0]
    dst = edge_index[1]
    zeros_n = jnp.zeros((N,), jnp.float32)
    zeros_nd = jnp.zeros((N, H), jnp.float32)

    deg = _degree_kernel(src, dst, zeros_n)
    gl, gr, d0 = _scale_call(x, deg.T)
    aggl, aggr = _gather_scatter_kernel(gl, gr, src, dst, zeros_nd)
    return _matmul_call(aggl, aggr, W_fc, d0)
